# baseline (speedup 1.0000x reference)
import functools

import jax
import jax.numpy as jnp
from jax import lax
from jax.experimental import pallas as pl
from jax.experimental.pallas import tpu as pltpu
from jax.experimental.pallas import tpu_sc as plsc

NC = 2
NS = 16
L = 16

K = 96
NBUF = 3
OWN = 624
ZROWS = 104


def _matmul_tc(features, wsplit):
    n, d_in = features.shape
    dh = wsplit.shape[2]
    blk = 1000

    def body(x_ref, w_ref, o_ref):
        o_ref[0] = jnp.dot(x_ref[...], w_ref[0],
                           preferred_element_type=jnp.float32)

    return pl.pallas_call(
        body,
        grid=(NC, n // blk),
        in_specs=[
            pl.BlockSpec((blk, d_in), lambda c, i: (i, 0)),
            pl.BlockSpec((1, d_in, dh), lambda c, i: (c, 0, 0)),
        ],
        out_specs=pl.BlockSpec((1, blk, dh), lambda c, i: (c, i, 0)),
        out_shape=jax.ShapeDtypeStruct((NC, n, dh), jnp.float32),
    )(features, wsplit)


def _combine_tc(partials, bsplit):
    _, n, dh = partials.shape
    blk = 1000

    def body(p_ref, b_ref, o_ref):
        o_ref[...] = jnp.concatenate(
            [p_ref[0] + b_ref[0], p_ref[1] + b_ref[1]], axis=1)

    return pl.pallas_call(
        body,
        grid=(n // blk,),
        in_specs=[
            pl.BlockSpec((NC, blk, dh), lambda i: (0, i, 0)),
            pl.BlockSpec((NC, 1, dh), lambda i: (0, 0, 0)),
        ],
        out_specs=pl.BlockSpec((blk, NC * dh), lambda i: (i, 0)),
        out_shape=jax.ShapeDtypeStruct((n, NC * dh), jnp.float32),
    )(partials, bsplit)


def _spmm_sc(hs, src, dst, ew, bsplit):
    _, n, dh = hs.shape
    nblk = src.shape[1]
    tail = n - OWN * NS

    mesh = plsc.VectorSubcoreMesh(core_axis_name="c", subcore_axis_name="s")

    @functools.partial(
        pl.kernel,
        out_type=jax.ShapeDtypeStruct((n, NC * dh), jnp.float32),
        mesh=mesh,
        scratch_types=[
            pltpu.VMEM((nblk, K), jnp.int32),
            pltpu.VMEM((nblk, K), jnp.int32),
            pltpu.VMEM((nblk, K), jnp.float32),
            pltpu.VMEM((NBUF, K, dh), jnp.float32),
            pltpu.VMEM((ZROWS, dh), jnp.float32),
            pltpu.VMEM((1, dh), jnp.float32),
            pltpu.VMEM_SHARED((n, dh), jnp.float32),
            pltpu.SemaphoreType.DMA((NBUF,)),
            pltpu.SemaphoreType.DMA((NBUF,)),
        ],
        compiler_params=pltpu.CompilerParams(use_tc_tiling_on_sc=False),
    )
    def sc_kernel(hs_hbm, src_hbm, dst_hbm, ew_hbm, b_hbm, out_hbm,
                  src_v, dst_v, ew_v, rows_v, zbuf, bias_v, acc,
                  gsems, ssems):
        cid = lax.axis_index("c")
        sid = lax.axis_index("s")

        pltpu.sync_copy(src_hbm.at[sid], src_v)
        pltpu.sync_copy(dst_hbm.at[sid], dst_v)
        pltpu.sync_copy(ew_hbm.at[sid], ew_v)
        pltpu.sync_copy(b_hbm.at[cid], bias_v)

        @pl.loop(0, ZROWS)
        def _(i):
            for q in range(dh // L):
                zbuf[i, pl.ds(q * L, L)] = bias_v[0, pl.ds(q * L, L)]

        base = pl.multiple_of(sid * OWN, 8)
        for z in range(OWN // ZROWS):
            pltpu.sync_copy(zbuf, acc.at[pl.ds(base + z * ZROWS, ZROWS)])

        @pl.when(sid == NS - 1)
        def _():
            pltpu.sync_copy(zbuf.at[pl.ds(0, tail)],
                            acc.at[pl.ds(n - tail, tail)])

        plsc.subcore_barrier()

        def gather(j, b):
            return pltpu.async_copy(hs_hbm.at[cid].at[src_v.at[j]],
                                    rows_v.at[b], gsems.at[b])

        def wait_gather(j, b):
            pltpu.make_async_copy(hs_hbm.at[cid].at[src_v.at[j]],
                                  rows_v.at[b], gsems.at[b]).wait()

        def scatter(j, b):
            return pltpu.async_copy(rows_v.at[b], acc.at[dst_v.at[j]],
                                    ssems.at[b], add=True)

        def wait_scatter(j, b):
            pltpu.make_async_copy(rows_v.at[b], acc.at[dst_v.at[j]],
                                  ssems.at[b]).wait()

        for p in range(NBUF - 1):
            gather(p, p)

        @pl.loop(0, nblk + NBUF - 1, step=NBUF)
        def _(jj):
            for b in range(NBUF):
                j = jj + b

                @pl.when(j < nblk)
                def _():
                    wait_gather(j, b)

                    @pl.when(j >= 1)
                    def _():
                        wait_scatter(j, (b - 1) % NBUF)

                    @pl.when(j + NBUF - 1 < nblk)
                    def _():
                        gather(j + NBUF - 1, (b - 1) % NBUF)

                    @plsc.parallel_loop(0, K // L)
                    def _(g):
                        wchunk = ew_v[j, pl.ds(g * L, L)]
                        for e in range(L):
                            i = g * L + e
                            w_s = wchunk[e]
                            for q in range(dh // L):
                                sl = pl.ds(q * L, L)
                                rows_v[b, i, sl] = rows_v[b, i, sl] * w_s

                    scatter(j, b)

        wait_scatter(nblk - 1, (nblk - 1) % NBUF)
        plsc.subcore_barrier()
        col = pl.multiple_of(cid * dh, 8)
        pltpu.sync_copy(acc.at[pl.ds(base, OWN)],
                        out_hbm.at[pl.ds(base, OWN), pl.ds(col, dh)])

        @pl.when(sid == NS - 1)
        def _():
            pltpu.sync_copy(acc.at[pl.ds(n - tail, tail)],
                            out_hbm.at[pl.ds(n - tail, tail), pl.ds(col, dh)])

    return sc_kernel(hs, src, dst, ew, bsplit)


def kernel(features, edge_index, edge_weight, weight, bias):
    e = edge_weight.shape[0]
    d_out = weight.shape[1]
    dh = d_out // NC
    nblk = -(-e // (NS * K))
    e_pad = NS * nblk * K
    padn = e_pad - e

    wsplit = jnp.stack([weight[:, :dh], weight[:, dh:]])
    bsplit = jnp.stack([bias[:, :dh], bias[:, dh:]])

    hs = _matmul_tc(features, wsplit)
    zi = jnp.zeros((padn,), jnp.int32)
    src = jnp.concatenate([edge_index[0], zi]).reshape(NS, nblk, K)
    dst = jnp.concatenate([edge_index[1], zi]).reshape(NS, nblk, K)
    ew = jnp.concatenate([edge_weight,
                          jnp.zeros((padn,), jnp.float32)]).reshape(NS, nblk, K)
    return _spmm_sc(hs, src, dst, ew, bsplit)


# speedup vs baseline: 1.1536x; 1.1536x over previous
import functools

import jax
import jax.numpy as jnp
from jax import lax
from jax.experimental import pallas as pl
from jax.experimental.pallas import tpu as pltpu
from jax.experimental.pallas import tpu_sc as plsc

NC = 2
NS = 16
L = 16

K = 80
NBUF = 3
OWN = 624
ZROWS = 208


def _matmul_tc(features, wsplit):
    n, d_in = features.shape
    dh = wsplit.shape[2]

    def body(x_ref, w_ref, o_ref):
        o_ref[0] = jnp.dot(x_ref[...], w_ref[0],
                           preferred_element_type=jnp.float32)

    return pl.pallas_call(
        body,
        grid=(NC,),
        in_specs=[
            pl.BlockSpec((n, d_in), lambda c: (0, 0)),
            pl.BlockSpec((1, d_in, dh), lambda c: (c, 0, 0)),
        ],
        out_specs=pl.BlockSpec((1, n, dh), lambda c: (c, 0, 0)),
        out_shape=jax.ShapeDtypeStruct((NC, n, dh), jnp.float32),
    )(features, wsplit)


def _combine_tc(partials, bsplit):
    _, n, dh = partials.shape
    blk = 1000

    def body(p_ref, b_ref, o_ref):
        o_ref[...] = jnp.concatenate(
            [p_ref[0] + b_ref[0], p_ref[1] + b_ref[1]], axis=1)

    return pl.pallas_call(
        body,
        grid=(n // blk,),
        in_specs=[
            pl.BlockSpec((NC, blk, dh), lambda i: (0, i, 0)),
            pl.BlockSpec((NC, 1, dh), lambda i: (0, 0, 0)),
        ],
        out_specs=pl.BlockSpec((blk, NC * dh), lambda i: (i, 0)),
        out_shape=jax.ShapeDtypeStruct((n, NC * dh), jnp.float32),
    )(partials, bsplit)


def _spmm_sc(hs, src, dst, ew, bsplit):
    _, n, dh = hs.shape
    nblk = src.shape[1]
    tail = n - OWN * NS

    mesh = plsc.VectorSubcoreMesh(core_axis_name="c", subcore_axis_name="s")

    @functools.partial(
        pl.kernel,
        out_type=jax.ShapeDtypeStruct((n, NC * dh), jnp.float32),
        mesh=mesh,
        scratch_types=[
            pltpu.VMEM((nblk, K), jnp.int32),
            pltpu.VMEM((nblk, K), jnp.int32),
            pltpu.VMEM((nblk, K), jnp.float32),
            pltpu.VMEM((NBUF, K, dh), jnp.float32),
            pltpu.VMEM((ZROWS, dh), jnp.float32),
            pltpu.VMEM((1, dh), jnp.float32),
            pltpu.VMEM_SHARED((n, dh), jnp.float32),
            pltpu.SemaphoreType.DMA((NBUF,)),
            pltpu.SemaphoreType.DMA((NBUF,)),
        ],
        compiler_params=pltpu.CompilerParams(use_tc_tiling_on_sc=False),
    )
    def sc_kernel(hs_hbm, src_hbm, dst_hbm, ew_hbm, b_hbm, out_hbm,
                  src_v, dst_v, ew_v, rows_v, zbuf, bias_v, acc,
                  gsems, ssems):
        cid = lax.axis_index("c")
        sid = lax.axis_index("s")

        pltpu.sync_copy(src_hbm.at[sid], src_v)
        pltpu.sync_copy(dst_hbm.at[sid], dst_v)
        pltpu.sync_copy(ew_hbm.at[sid], ew_v)
        pltpu.sync_copy(b_hbm.at[cid], bias_v)

        @pl.loop(0, ZROWS)
        def _(i):
            for q in range(dh // L):
                zbuf[i, pl.ds(q * L, L)] = bias_v[0, pl.ds(q * L, L)]

        base = pl.multiple_of(sid * OWN, 8)
        for z in range(OWN // ZROWS):
            pltpu.sync_copy(zbuf, acc.at[pl.ds(base + z * ZROWS, ZROWS)])

        @pl.when(sid == NS - 1)
        def _():
            pltpu.sync_copy(zbuf.at[pl.ds(0, tail)],
                            acc.at[pl.ds(n - tail, tail)])

        plsc.subcore_barrier()

        def gather(j, b):
            return pltpu.async_copy(hs_hbm.at[cid].at[src_v.at[j]],
                                    rows_v.at[b], gsems.at[b])

        def wait_gather(j, b):
            pltpu.make_async_copy(hs_hbm.at[cid].at[src_v.at[j]],
                                  rows_v.at[b], gsems.at[b]).wait()

        def scatter(j, b):
            return pltpu.async_copy(rows_v.at[b], acc.at[dst_v.at[j]],
                                    ssems.at[b], add=True)

        def wait_scatter(j, b):
            pltpu.make_async_copy(rows_v.at[b], acc.at[dst_v.at[j]],
                                  ssems.at[b]).wait()

        for p in range(NBUF - 1):
            gather(p, p)

        @pl.loop(0, nblk + NBUF - 1, step=NBUF)
        def _(jj):
            for b in range(NBUF):
                j = jj + b

                @pl.when(j < nblk)
                def _():
                    wait_gather(j, b)

                    @pl.when(j >= 1)
                    def _():
                        wait_scatter(j, (b - 1) % NBUF)

                    @pl.when(j + NBUF - 1 < nblk)
                    def _():
                        gather(j + NBUF - 1, (b - 1) % NBUF)

                    @plsc.parallel_loop(0, K // L)
                    def _(g):
                        wchunk = ew_v[j, pl.ds(g * L, L)]
                        for e in range(L):
                            i = g * L + e
                            w_s = wchunk[e]
                            for q in range(dh // L):
                                sl = pl.ds(q * L, L)
                                rows_v[b, i, sl] = rows_v[b, i, sl] * w_s

                    scatter(j, b)

        wait_scatter(nblk - 1, (nblk - 1) % NBUF)
        plsc.subcore_barrier()
        col = pl.multiple_of(cid * dh, 8)
        pltpu.sync_copy(acc.at[pl.ds(base, OWN)],
                        out_hbm.at[pl.ds(base, OWN), pl.ds(col, dh)])

        @pl.when(sid == NS - 1)
        def _():
            pltpu.sync_copy(acc.at[pl.ds(n - tail, tail)],
                            out_hbm.at[pl.ds(n - tail, tail), pl.ds(col, dh)])

    return sc_kernel(hs, src, dst, ew, bsplit)


def kernel(features, edge_index, edge_weight, weight, bias):
    e = edge_weight.shape[0]
    d_out = weight.shape[1]
    dh = d_out // NC
    nblk = -(-e // (NS * K))
    e_pad = NS * nblk * K
    padn = e_pad - e

    wsplit = jnp.stack([weight[:, :dh], weight[:, dh:]])
    bsplit = jnp.stack([bias[:, :dh], bias[:, dh:]])

    hs = _matmul_tc(features, wsplit)
    zi = jnp.zeros((padn,), jnp.int32)
    src = jnp.concatenate([edge_index[0], zi]).reshape(NS, nblk, K)
    dst = jnp.concatenate([edge_index[1], zi]).reshape(NS, nblk, K)
    ew = jnp.concatenate([edge_weight,
                          jnp.zeros((padn,), jnp.float32)]).reshape(NS, nblk, K)
    return _spmm_sc(hs, src, dst, ew, bsplit)
